# baseline (speedup 1.0000x reference)
import functools

import jax
import jax.numpy as jnp
from jax import lax
from jax.experimental import pallas as pl
from jax.experimental.pallas import tpu as pltpu
from jax.experimental.pallas import tpu_sc as plsc

NN = 10000
EE = 320000
DD = 128
NG = 8
NCLS = 32
NCORES = 2
NSUB = 16
NWORK = NCORES * NSUB
CH = 80
EPW = EE // NWORK
NCH = EPW // CH
ZR = 624
ZREM = NN - NSUB * ZR
BR = 1000

_MESH = plsc.VectorSubcoreMesh(core_axis_name="c", subcore_axis_name="s")


CH2 = 40
NCH2 = EPW // CH2
ROT = 5
GRPS = NCH2 // ROT

@functools.partial(
    pl.kernel,
    mesh=_MESH,
    out_type=jax.ShapeDtypeStruct((NCORES, NN, DD), jnp.float32),
    scratch_types=[
        pltpu.VMEM((EPW,), jnp.int32),
    ] + [pltpu.VMEM((CH2,), jnp.int32) for _ in range(ROT)]
      + [pltpu.VMEM((CH2, DD), jnp.float32) for _ in range(ROT)]
      + [pltpu.VMEM_SHARED((NN, DD), jnp.float32)]
      + [pltpu.SemaphoreType.DMA for _ in range(3 * ROT)],
)
def _sc_prop(y_hbm, row_hbm, col_hbm, out_hbm, ridx1d, *rest):
    cbufs = rest[:ROT]
    bufs = rest[ROT:2 * ROT]
    acc = rest[2 * ROT]
    semg = rest[2 * ROT + 1:3 * ROT + 1]
    sems = rest[3 * ROT + 1:4 * ROT + 1]
    semi = rest[4 * ROT + 1:5 * ROT + 1]

    c = lax.axis_index("c")
    s = lax.axis_index("s")
    wid = s * NCORES + c
    base = wid * EPW

    def zrow(i, carry):
        for k in range(DD // 16):
            bufs[0][i, pl.ds(k * 16, 16)] = jnp.zeros((16,), jnp.float32)
        return carry

    lax.fori_loop(0, CH2, zrow, 0)
    for q in range(ZR // CH2):
        pltpu.sync_copy(bufs[0], acc.at[pl.ds(s * ZR + q * CH2, CH2)])
    pltpu.sync_copy(bufs[0].at[pl.ds(0, ZR % CH2)],
                    acc.at[pl.ds(s * ZR + (ZR // CH2) * CH2, ZR % CH2)])

    @pl.when(s == NSUB - 1)
    def _zero_rem():
        pltpu.sync_copy(bufs[0].at[pl.ds(0, ZREM)],
                        acc.at[pl.ds(NSUB * ZR, ZREM)])

    plsc.subcore_barrier()

    pltpu.sync_copy(row_hbm.at[pl.ds(base, EPW)], ridx1d)

    for j0 in range(2):
        pltpu.async_copy(col_hbm.at[pl.ds(base + j0 * CH2, CH2)],
                         cbufs[j0], semi[j0])
        pltpu.async_copy(y_hbm.at[ridx1d.at[pl.ds(j0 * CH2, CH2)]],
                         bufs[j0], semg[j0])

    def grp(g, carry):
        for b in range(ROT):
            j = g * ROT + b
            r2 = (b + 2) % ROT

            @pl.when(j + 2 < NCH2)
            def _prefetch():
                @pl.when(j >= ROT - 2)
                def _drain_scatter():
                    pltpu.make_async_copy(
                        bufs[r2], acc.at[cbufs[r2]], sems[r2]).wait()

                pltpu.async_copy(col_hbm.at[pl.ds(base + (j + 2) * CH2, CH2)],
                                 cbufs[r2], semi[r2])
                pltpu.async_copy(
                    y_hbm.at[ridx1d.at[pl.ds((j + 2) * CH2, CH2)]],
                    bufs[r2], semg[r2])

            pltpu.make_async_copy(
                col_hbm.at[pl.ds(base, CH2)], cbufs[b], semi[b]).wait()
            pltpu.make_async_copy(
                y_hbm.at[ridx1d.at[pl.ds(0, CH2)]], bufs[b], semg[b]).wait()
            pltpu.async_copy(bufs[b], acc.at[cbufs[b]], sems[b], add=True)
        return carry

    lax.fori_loop(0, GRPS, grp, 0)

    for b in range(ROT):
        pltpu.make_async_copy(bufs[b], acc.at[cbufs[b]], sems[b]).wait()

    plsc.subcore_barrier()
    pltpu.sync_copy(acc.at[pl.ds(s * ZR, ZR)],
                    out_hbm.at[c, pl.ds(s * ZR, ZR)])

    @pl.when(s == NSUB - 1)
    def _dump_rem():
        pltpu.sync_copy(acc.at[pl.ds(NSUB * ZR, ZREM)],
                        out_hbm.at[c, pl.ds(NSUB * ZR, ZREM)])


def _dis_body(degp_ref, dis_ref):
    t = degp_ref[...]
    deg = t[0, :, 0] + t[1, :, 0]
    pos = deg > 0.0
    inv = jnp.where(pos, lax.rsqrt(jnp.maximum(deg, 1e-12)), 0.0)
    dis_ref[...] = jnp.stack(
        [inv, inv * inv, jnp.where(pos, jnp.sqrt(deg), 0.0)], axis=0)


def _dis_call(degp):
    return pl.pallas_call(
        _dis_body,
        out_shape=jax.ShapeDtypeStruct((3, NN), jnp.float32),
    )(degp)


def _ta_body(h_ref, dis_ref, y_ref):
    y_ref[...] = jnp.maximum(h_ref[...], 0.0) * dis_ref[...]


def _ta_call(h, dis):
    grid = NN // BR
    return pl.pallas_call(
        _ta_body,
        grid=(grid,),
        in_specs=[
            pl.BlockSpec((BR, DD), lambda i: (i, 0)),
            pl.BlockSpec((BR, 1), lambda i: (i, 0)),
        ],
        out_specs=pl.BlockSpec((BR, DD), lambda i: (i, 0)),
        out_shape=jax.ShapeDtypeStruct((NN, DD), jnp.float32),
    )(h, dis)


def _tb_body(p0_ref, p1_ref, dis2_ref, y_ref):
    y_ref[...] = (p0_ref[...] + p1_ref[...]) * dis2_ref[...]


def _tb_call(p0, p1, dis2):
    grid = NN // BR
    return pl.pallas_call(
        _tb_body,
        grid=(grid,),
        in_specs=[
            pl.BlockSpec((BR, DD), lambda i: (i, 0)),
            pl.BlockSpec((BR, DD), lambda i: (i, 0)),
            pl.BlockSpec((BR, 1), lambda i: (i, 0)),
        ],
        out_specs=pl.BlockSpec((BR, DD), lambda i: (i, 0)),
        out_shape=jax.ShapeDtypeStruct((NN, DD), jnp.float32),
    )(p0, p1, dis2)


def _te_body(y0_ref, y1_ref, y2_ref, y3_ref, disr_ref, w_ref, b_ref, out_ref):
    dr = disr_ref[...]
    w = w_ref[...]
    acc = jnp.dot(y0_ref[...] * dr, w[0], preferred_element_type=jnp.float32)
    acc += jnp.dot(y1_ref[...] * dr, w[1], preferred_element_type=jnp.float32)
    acc += jnp.dot(y2_ref[...] * dr, w[2], preferred_element_type=jnp.float32)
    acc += jnp.dot(y3_ref[...] * dr, w[3], preferred_element_type=jnp.float32)
    out_ref[...] = acc + b_ref[...]


def _te_call(y0, y1, y2, y3, disr, w, b):
    grid = NN // BR
    yspec = pl.BlockSpec((BR, DD), lambda i: (i, 0))
    return pl.pallas_call(
        _te_body,
        grid=(grid,),
        in_specs=[
            yspec, yspec, yspec, yspec,
            pl.BlockSpec((BR, 1), lambda i: (i, 0)),
            pl.BlockSpec((4, DD, DD), lambda i: (0, 0, 0)),
            pl.BlockSpec((1, DD), lambda i: (0, 0)),
        ],
        out_specs=pl.BlockSpec((BR, DD), lambda i: (i, 0)),
        out_shape=jax.ShapeDtypeStruct((NN, DD), jnp.float32),
    )(y0, y1, y2, y3, disr, w, b)


def _pool_body(h_ref, oh_ref, w0_ref, b0_ref, w1_ref, b1_ref, out_ref):
    h = h_ref[...]
    oh = oh_ref[...]
    sums = lax.dot_general(oh, h, (((0,), (0,)), ((), ())),
                           preferred_element_type=jnp.float32)
    cnt = jnp.sum(oh, axis=0)
    mean = sums / jnp.maximum(cnt, 1.0)[:, None]
    mx_rows = []
    for g in range(NG):
        m = oh[:, g:g + 1] > 0.5
        mx_rows.append(jnp.max(jnp.where(m, h, -jnp.inf), axis=0)[None])
    gmax = jnp.concatenate(mx_rows, axis=0)
    gcat = jnp.concatenate([mean, gmax], axis=1)
    gr = jnp.maximum(gcat, 0.0)
    a1 = jnp.maximum(
        jnp.dot(gr, w0_ref[...], preferred_element_type=jnp.float32)
        + b0_ref[...], 0.0)
    out_ref[...] = jnp.dot(
        a1, w1_ref[...], preferred_element_type=jnp.float32) + b1_ref[...]


def _pool_call(h, onehot, w0, b0, w1, b1):
    return pl.pallas_call(
        _pool_body,
        out_shape=jax.ShapeDtypeStruct((NG, NCLS), jnp.float32),
    )(h, onehot, w0, b0, w1, b1)


def kernel(x, edge_index, batch, conv0_w, conv0_b, conv1_w, conv1_b,
           mlp0_w, mlp0_b, pred_w, pred_b):
    row = edge_index[0]
    col = edge_index[1]

    ones_tab = jnp.ones((NN, DD), jnp.float32)
    rows_lin = jnp.tile(jnp.arange(EPW, dtype=jnp.int32), NWORK)
    degp = _sc_prop(ones_tab, rows_lin, col)
    dd = _dis_call(degp)
    dis = dd[0].reshape(NN, 1)
    dis2 = dd[1].reshape(NN, 1)
    disr = dd[2].reshape(NN, 1)

    onehot = (batch[:, None] == jnp.arange(NG, dtype=batch.dtype)
              ).astype(jnp.float32)

    h = x
    for layer in range(2):
        w = conv0_w if layer == 0 else conv1_w
        b = conv0_b if layer == 0 else conv1_b
        ys = [_ta_call(h, dis)]
        for k in (1, 2, 3):
            pp = _sc_prop(ys[-1], row, col)
            ys.append(_tb_call(pp[0], pp[1], dis2))
        h = _te_call(ys[0], ys[1], ys[2], ys[3], disr, w,
                     b.reshape(1, DD))

    return _pool_call(h, onehot, mlp0_w, mlp0_b.reshape(1, 2 * DD),
                      pred_w, pred_b.reshape(1, NCLS))


# speedup vs baseline: 1.0333x; 1.0333x over previous
import functools

import jax
import jax.numpy as jnp
from jax import lax
from jax.experimental import pallas as pl
from jax.experimental.pallas import tpu as pltpu
from jax.experimental.pallas import tpu_sc as plsc

NN = 10000
EE = 320000
DD = 128
NG = 8
NCLS = 32
NCORES = 2
NSUB = 16
NWORK = NCORES * NSUB
CH = 80
EPW = EE // NWORK
NCH = EPW // CH
ZR = 624
ZREM = NN - NSUB * ZR
BR = 1000

_MESH = plsc.VectorSubcoreMesh(core_axis_name="c", subcore_axis_name="s")


CH2 = 40
NCH2 = EPW // CH2
ROT = 5
GRPS = NCH2 // ROT

@functools.partial(
    pl.kernel,
    mesh=_MESH,
    out_type=jax.ShapeDtypeStruct((NCORES, NN, DD), jnp.float32),
    scratch_types=[
        pltpu.VMEM((EPW,), jnp.int32),
    ] + [pltpu.VMEM((CH2,), jnp.int32) for _ in range(ROT)]
      + [pltpu.VMEM((CH2, DD), jnp.float32) for _ in range(ROT)]
      + [pltpu.VMEM_SHARED((NN, DD), jnp.float32)]
      + [pltpu.SemaphoreType.DMA for _ in range(3 * ROT)],
)
def _sc_prop(y_hbm, row_hbm, col_hbm, out_hbm, ridx1d, *rest):
    cbufs = rest[:ROT]
    bufs = rest[ROT:2 * ROT]
    acc = rest[2 * ROT]
    semg = rest[2 * ROT + 1:3 * ROT + 1]
    sems = rest[3 * ROT + 1:4 * ROT + 1]
    semi = rest[4 * ROT + 1:5 * ROT + 1]

    c = lax.axis_index("c")
    s = lax.axis_index("s")
    wid = s * NCORES + c
    base = wid * EPW

    def zrow(i, carry):
        for k in range(DD // 16):
            bufs[0][i, pl.ds(k * 16, 16)] = jnp.zeros((16,), jnp.float32)
        return carry

    lax.fori_loop(0, CH2, zrow, 0)
    for q in range(ZR // CH2):
        pltpu.sync_copy(bufs[0], acc.at[pl.ds(s * ZR + q * CH2, CH2)])
    pltpu.sync_copy(bufs[0].at[pl.ds(0, ZR % CH2)],
                    acc.at[pl.ds(s * ZR + (ZR // CH2) * CH2, ZR % CH2)])

    @pl.when(s == NSUB - 1)
    def _zero_rem():
        pltpu.sync_copy(bufs[0].at[pl.ds(0, ZREM)],
                        acc.at[pl.ds(NSUB * ZR, ZREM)])

    plsc.subcore_barrier()

    pltpu.sync_copy(row_hbm.at[pl.ds(base, EPW)], ridx1d)

    for j0 in range(2):
        pltpu.async_copy(col_hbm.at[pl.ds(base + j0 * CH2, CH2)],
                         cbufs[j0], semi[j0])
        pltpu.async_copy(y_hbm.at[ridx1d.at[pl.ds(j0 * CH2, CH2)]],
                         bufs[j0], semg[j0])

    def grp(g, carry):
        for b in range(ROT):
            j = g * ROT + b
            r2 = (b + 2) % ROT

            @pl.when(j + 2 < NCH2)
            def _prefetch():
                @pl.when(j >= ROT - 2)
                def _drain_scatter():
                    pltpu.make_async_copy(
                        bufs[r2], acc.at[cbufs[r2]], sems[r2]).wait()

                pltpu.async_copy(col_hbm.at[pl.ds(base + (j + 2) * CH2, CH2)],
                                 cbufs[r2], semi[r2])
                pltpu.async_copy(
                    y_hbm.at[ridx1d.at[pl.ds((j + 2) * CH2, CH2)]],
                    bufs[r2], semg[r2])

            pltpu.make_async_copy(
                col_hbm.at[pl.ds(base, CH2)], cbufs[b], semi[b]).wait()
            pltpu.make_async_copy(
                y_hbm.at[ridx1d.at[pl.ds(0, CH2)]], bufs[b], semg[b]).wait()
            pltpu.async_copy(bufs[b], acc.at[cbufs[b]], sems[b], add=True)
        return carry

    lax.fori_loop(0, GRPS, grp, 0)

    for b in range(ROT):
        pltpu.make_async_copy(bufs[b], acc.at[cbufs[b]], sems[b]).wait()

    plsc.subcore_barrier()
    pltpu.sync_copy(acc.at[pl.ds(s * ZR, ZR)],
                    out_hbm.at[c, pl.ds(s * ZR, ZR)])

    @pl.when(s == NSUB - 1)
    def _dump_rem():
        pltpu.sync_copy(acc.at[pl.ds(NSUB * ZR, ZREM)],
                        out_hbm.at[c, pl.ds(NSUB * ZR, ZREM)])


def _head_body(degp_ref, x_ref, dis_ref, dis2_ref, disr_ref, y_ref):
    t = degp_ref[...]
    deg = t[0, :, 0] + t[1, :, 0]
    pos = deg > 0.0
    inv = jnp.where(pos, lax.rsqrt(jnp.maximum(deg, 1e-12)), 0.0)
    dis_ref[...] = inv[:, None]
    dis2_ref[...] = (inv * inv)[:, None]
    disr_ref[...] = jnp.where(pos, jnp.sqrt(deg), 0.0)[:, None]
    y_ref[...] = jnp.maximum(x_ref[...], 0.0) * inv[:, None]


def _head_call(degp, x):
    grid = NN // BR
    return pl.pallas_call(
        _head_body,
        grid=(grid,),
        in_specs=[
            pl.BlockSpec((NCORES, BR, DD), lambda i: (0, i, 0)),
            pl.BlockSpec((BR, DD), lambda i: (i, 0)),
        ],
        out_specs=[
            pl.BlockSpec((BR, 1), lambda i: (i, 0)),
            pl.BlockSpec((BR, 1), lambda i: (i, 0)),
            pl.BlockSpec((BR, 1), lambda i: (i, 0)),
            pl.BlockSpec((BR, DD), lambda i: (i, 0)),
        ],
        out_shape=[
            jax.ShapeDtypeStruct((NN, 1), jnp.float32),
            jax.ShapeDtypeStruct((NN, 1), jnp.float32),
            jax.ShapeDtypeStruct((NN, 1), jnp.float32),
            jax.ShapeDtypeStruct((NN, DD), jnp.float32),
        ],
    )(degp, x)


def _tb_body(p0_ref, p1_ref, dis2_ref, y_ref):
    y_ref[...] = (p0_ref[...] + p1_ref[...]) * dis2_ref[...]


def _tb_call(p0, p1, dis2):
    grid = NN // BR
    return pl.pallas_call(
        _tb_body,
        grid=(grid,),
        in_specs=[
            pl.BlockSpec((BR, DD), lambda i: (i, 0)),
            pl.BlockSpec((BR, DD), lambda i: (i, 0)),
            pl.BlockSpec((BR, 1), lambda i: (i, 0)),
        ],
        out_specs=pl.BlockSpec((BR, DD), lambda i: (i, 0)),
        out_shape=jax.ShapeDtypeStruct((NN, DD), jnp.float32),
    )(p0, p1, dis2)


def _mid_body(y0_ref, y1_ref, y2_ref, y3_ref, disr_ref, dis_ref, w_ref,
              b_ref, out_ref):
    dr = disr_ref[...]
    w = w_ref[...]
    acc = jnp.dot(y0_ref[...] * dr, w[0], preferred_element_type=jnp.float32)
    acc += jnp.dot(y1_ref[...] * dr, w[1], preferred_element_type=jnp.float32)
    acc += jnp.dot(y2_ref[...] * dr, w[2], preferred_element_type=jnp.float32)
    acc += jnp.dot(y3_ref[...] * dr, w[3], preferred_element_type=jnp.float32)
    out_ref[...] = jnp.maximum(acc + b_ref[...], 0.0) * dis_ref[...]


def _mid_call(y0, y1, y2, y3, disr, dis, w, b):
    grid = NN // BR
    yspec = pl.BlockSpec((BR, DD), lambda i: (i, 0))
    cspec = pl.BlockSpec((BR, 1), lambda i: (i, 0))
    return pl.pallas_call(
        _mid_body,
        grid=(grid,),
        in_specs=[
            yspec, yspec, yspec, yspec, cspec, cspec,
            pl.BlockSpec((4, DD, DD), lambda i: (0, 0, 0)),
            pl.BlockSpec((1, DD), lambda i: (0, 0)),
        ],
        out_specs=yspec,
        out_shape=jax.ShapeDtypeStruct((NN, DD), jnp.float32),
    )(y0, y1, y2, y3, disr, dis, w, b)


def _tail_body(y0_ref, y1_ref, y2_ref, y3_ref, disr_ref, w_ref, b_ref,
               oh_ref, w0_ref, b0_ref, w1_ref, b1_ref, out_ref,
               sum_s, mx_s, cnt_s):
    i = pl.program_id(0)

    @pl.when(i == 0)
    def _init():
        sum_s[...] = jnp.zeros((NG, DD), jnp.float32)
        mx_s[...] = jnp.full((NG, DD), -jnp.inf, jnp.float32)
        cnt_s[...] = jnp.zeros((NG, DD), jnp.float32)

    dr = disr_ref[...]
    w = w_ref[...]
    h = jnp.dot(y0_ref[...] * dr, w[0], preferred_element_type=jnp.float32)
    h += jnp.dot(y1_ref[...] * dr, w[1], preferred_element_type=jnp.float32)
    h += jnp.dot(y2_ref[...] * dr, w[2], preferred_element_type=jnp.float32)
    h += jnp.dot(y3_ref[...] * dr, w[3], preferred_element_type=jnp.float32)
    h += b_ref[...]
    oh = oh_ref[...]
    sum_s[...] += lax.dot_general(oh, h, (((0,), (0,)), ((), ())),
                                  preferred_element_type=jnp.float32)
    cnt_s[...] += jnp.sum(oh, axis=0)[:, None]
    mx = mx_s[...]
    mx_rows = []
    for g in range(NG):
        m = oh[:, g:g + 1] > 0.5
        mx_rows.append(jnp.max(jnp.where(m, h, -jnp.inf), axis=0)[None])
    mx_s[...] = jnp.maximum(mx, jnp.concatenate(mx_rows, axis=0))

    @pl.when(i == NN // BR - 1)
    def _finish():
        mean = sum_s[...] / jnp.maximum(cnt_s[...], 1.0)
        gcat = jnp.concatenate([mean, mx_s[...]], axis=1)
        gr = jnp.maximum(gcat, 0.0)
        a1 = jnp.maximum(
            jnp.dot(gr, w0_ref[...], preferred_element_type=jnp.float32)
            + b0_ref[...], 0.0)
        out_ref[...] = jnp.dot(
            a1, w1_ref[...], preferred_element_type=jnp.float32) + b1_ref[...]


def _tail_call(y0, y1, y2, y3, disr, w, b, onehot, w0, b0, w1, b1):
    grid = NN // BR
    yspec = pl.BlockSpec((BR, DD), lambda i: (i, 0))
    return pl.pallas_call(
        _tail_body,
        grid=(grid,),
        in_specs=[
            yspec, yspec, yspec, yspec,
            pl.BlockSpec((BR, 1), lambda i: (i, 0)),
            pl.BlockSpec((4, DD, DD), lambda i: (0, 0, 0)),
            pl.BlockSpec((1, DD), lambda i: (0, 0)),
            pl.BlockSpec((BR, NG), lambda i: (i, 0)),
            pl.BlockSpec((2 * DD, 2 * DD), lambda i: (0, 0)),
            pl.BlockSpec((1, 2 * DD), lambda i: (0, 0)),
            pl.BlockSpec((2 * DD, NCLS), lambda i: (0, 0)),
            pl.BlockSpec((1, NCLS), lambda i: (0, 0)),
        ],
        out_specs=pl.BlockSpec((NG, NCLS), lambda i: (0, 0)),
        out_shape=jax.ShapeDtypeStruct((NG, NCLS), jnp.float32),
        scratch_shapes=[
            pltpu.VMEM((NG, DD), jnp.float32),
            pltpu.VMEM((NG, DD), jnp.float32),
            pltpu.VMEM((NG, DD), jnp.float32),
        ],
    )(y0, y1, y2, y3, disr, w, b, onehot, w0, b0, w1, b1)


def kernel(x, edge_index, batch, conv0_w, conv0_b, conv1_w, conv1_b,
           mlp0_w, mlp0_b, pred_w, pred_b):
    row = edge_index[0]
    col = edge_index[1]

    ones_tab = jnp.ones((NN, DD), jnp.float32)
    rows_lin = jnp.tile(jnp.arange(EPW, dtype=jnp.int32), NWORK)
    degp = _sc_prop(ones_tab, rows_lin, col)

    dis, dis2, disr, y = _head_call(degp, x)

    onehot = (batch[:, None] == jnp.arange(NG, dtype=batch.dtype)
              ).astype(jnp.float32)

    ys = [y]
    for k in (1, 2, 3):
        pp = _sc_prop(ys[-1], row, col)
        ys.append(_tb_call(pp[0], pp[1], dis2))
    y = _mid_call(ys[0], ys[1], ys[2], ys[3], disr, dis, conv0_w,
                  conv0_b.reshape(1, DD))

    ys = [y]
    for k in (1, 2, 3):
        pp = _sc_prop(ys[-1], row, col)
        ys.append(_tb_call(pp[0], pp[1], dis2))
    return _tail_call(ys[0], ys[1], ys[2], ys[3], disr, conv1_w,
                      conv1_b.reshape(1, DD), onehot, mlp0_w,
                      mlp0_b.reshape(1, 2 * DD), pred_w,
                      pred_b.reshape(1, NCLS))


# speedup vs baseline: 1.0335x; 1.0002x over previous
import functools

import jax
import jax.numpy as jnp
from jax import lax
from jax.experimental import pallas as pl
from jax.experimental.pallas import tpu as pltpu
from jax.experimental.pallas import tpu_sc as plsc

NN = 10000
EE = 320000
DD = 128
NG = 8
NCLS = 32
NCORES = 2
NSUB = 16
NWORK = NCORES * NSUB
EPW = EE // NWORK
ZR = 624
ZREM = NN - NSUB * ZR
BR = 1000

_MESH = plsc.VectorSubcoreMesh(core_axis_name="c", subcore_axis_name="s")


CH2 = 40
NCH2 = EPW // CH2
ROT = 5
GRPS = NCH2 // ROT

@functools.partial(
    pl.kernel,
    mesh=_MESH,
    out_type=jax.ShapeDtypeStruct((NCORES, NN, DD), jnp.float32),
    scratch_types=[
        pltpu.VMEM((EPW,), jnp.int32),
    ] + [pltpu.VMEM((CH2,), jnp.int32) for _ in range(ROT)]
      + [pltpu.VMEM((CH2, DD), jnp.float32) for _ in range(ROT)]
      + [pltpu.VMEM_SHARED((NN, DD), jnp.float32)]
      + [pltpu.SemaphoreType.DMA for _ in range(3 * ROT)],
)
def _sc_prop(y_hbm, row_hbm, col_hbm, out_hbm, ridx1d, *rest):
    cbufs = rest[:ROT]
    bufs = rest[ROT:2 * ROT]
    acc = rest[2 * ROT]
    semg = rest[2 * ROT + 1:3 * ROT + 1]
    sems = rest[3 * ROT + 1:4 * ROT + 1]
    semi = rest[4 * ROT + 1:5 * ROT + 1]

    c = lax.axis_index("c")
    s = lax.axis_index("s")
    wid = s * NCORES + c
    base = wid * EPW

    def zrow(i, carry):
        for k in range(DD // 16):
            bufs[0][i, pl.ds(k * 16, 16)] = jnp.zeros((16,), jnp.float32)
        return carry

    lax.fori_loop(0, CH2, zrow, 0)
    for q in range(ZR // CH2):
        pltpu.sync_copy(bufs[0], acc.at[pl.ds(s * ZR + q * CH2, CH2)])
    pltpu.sync_copy(bufs[0].at[pl.ds(0, ZR % CH2)],
                    acc.at[pl.ds(s * ZR + (ZR // CH2) * CH2, ZR % CH2)])

    @pl.when(s == NSUB - 1)
    def _zero_rem():
        pltpu.sync_copy(bufs[0].at[pl.ds(0, ZREM)],
                        acc.at[pl.ds(NSUB * ZR, ZREM)])

    plsc.subcore_barrier()

    pltpu.sync_copy(row_hbm.at[pl.ds(base, EPW)], ridx1d)

    for j0 in range(2):
        pltpu.async_copy(col_hbm.at[pl.ds(base + j0 * CH2, CH2)],
                         cbufs[j0], semi[j0])
        pltpu.async_copy(y_hbm.at[ridx1d.at[pl.ds(j0 * CH2, CH2)]],
                         bufs[j0], semg[j0])

    def grp(g, carry):
        for b in range(ROT):
            j = g * ROT + b
            r2 = (b + 2) % ROT

            @pl.when(j + 2 < NCH2)
            def _prefetch():
                @pl.when(j >= ROT - 2)
                def _drain_scatter():
                    pltpu.make_async_copy(
                        bufs[r2], acc.at[cbufs[r2]], sems[r2]).wait()

                pltpu.async_copy(col_hbm.at[pl.ds(base + (j + 2) * CH2, CH2)],
                                 cbufs[r2], semi[r2])
                pltpu.async_copy(
                    y_hbm.at[ridx1d.at[pl.ds((j + 2) * CH2, CH2)]],
                    bufs[r2], semg[r2])

            pltpu.make_async_copy(
                col_hbm.at[pl.ds(base, CH2)], cbufs[b], semi[b]).wait()
            pltpu.make_async_copy(
                y_hbm.at[ridx1d.at[pl.ds(0, CH2)]], bufs[b], semg[b]).wait()
            pltpu.async_copy(bufs[b], acc.at[cbufs[b]], sems[b], add=True)
        return carry

    lax.fori_loop(0, GRPS, grp, 0)

    for b in range(ROT):
        pltpu.make_async_copy(bufs[b], acc.at[cbufs[b]], sems[b]).wait()

    plsc.subcore_barrier()
    pltpu.sync_copy(acc.at[pl.ds(s * ZR, ZR)],
                    out_hbm.at[c, pl.ds(s * ZR, ZR)])

    @pl.when(s == NSUB - 1)
    def _dump_rem():
        pltpu.sync_copy(acc.at[pl.ds(NSUB * ZR, ZREM)],
                        out_hbm.at[c, pl.ds(NSUB * ZR, ZREM)])


def _head_body(degp_ref, x_ref, dis_ref, dis2_ref, disr_ref, y_ref):
    t = degp_ref[...]
    deg = t[0, :, 0] + t[1, :, 0]
    pos = deg > 0.0
    inv = jnp.where(pos, lax.rsqrt(jnp.maximum(deg, 1e-12)), 0.0)
    dis_ref[...] = inv[:, None]
    dis2_ref[...] = (inv * inv)[:, None]
    disr_ref[...] = jnp.where(pos, jnp.sqrt(deg), 0.0)[:, None]
    y_ref[...] = jnp.maximum(x_ref[...], 0.0) * inv[:, None]


def _head_call(degp, x):
    grid = NN // BR
    return pl.pallas_call(
        _head_body,
        grid=(grid,),
        in_specs=[
            pl.BlockSpec((NCORES, BR, DD), lambda i: (0, i, 0)),
            pl.BlockSpec((BR, DD), lambda i: (i, 0)),
        ],
        out_specs=[
            pl.BlockSpec((BR, 1), lambda i: (i, 0)),
            pl.BlockSpec((BR, 1), lambda i: (i, 0)),
            pl.BlockSpec((BR, 1), lambda i: (i, 0)),
            pl.BlockSpec((BR, DD), lambda i: (i, 0)),
        ],
        out_shape=[
            jax.ShapeDtypeStruct((NN, 1), jnp.float32),
            jax.ShapeDtypeStruct((NN, 1), jnp.float32),
            jax.ShapeDtypeStruct((NN, 1), jnp.float32),
            jax.ShapeDtypeStruct((NN, DD), jnp.float32),
        ],
    )(degp, x)


def _tb_body(p0_ref, p1_ref, dis2_ref, y_ref):
    y_ref[...] = (p0_ref[...] + p1_ref[...]) * dis2_ref[...]


def _tb_call(p0, p1, dis2):
    grid = NN // BR
    return pl.pallas_call(
        _tb_body,
        grid=(grid,),
        in_specs=[
            pl.BlockSpec((BR, DD), lambda i: (i, 0)),
            pl.BlockSpec((BR, DD), lambda i: (i, 0)),
            pl.BlockSpec((BR, 1), lambda i: (i, 0)),
        ],
        out_specs=pl.BlockSpec((BR, DD), lambda i: (i, 0)),
        out_shape=jax.ShapeDtypeStruct((NN, DD), jnp.float32),
    )(p0, p1, dis2)


def _mid_body(y0_ref, y1_ref, y2_ref, y3_ref, disr_ref, dis_ref, w_ref,
              b_ref, out_ref):
    dr = disr_ref[...]
    w = w_ref[...]
    acc = jnp.dot(y0_ref[...] * dr, w[0], preferred_element_type=jnp.float32)
    acc += jnp.dot(y1_ref[...] * dr, w[1], preferred_element_type=jnp.float32)
    acc += jnp.dot(y2_ref[...] * dr, w[2], preferred_element_type=jnp.float32)
    acc += jnp.dot(y3_ref[...] * dr, w[3], preferred_element_type=jnp.float32)
    out_ref[...] = jnp.maximum(acc + b_ref[...], 0.0) * dis_ref[...]


def _mid_call(y0, y1, y2, y3, disr, dis, w, b):
    grid = NN // BR
    yspec = pl.BlockSpec((BR, DD), lambda i: (i, 0))
    cspec = pl.BlockSpec((BR, 1), lambda i: (i, 0))
    return pl.pallas_call(
        _mid_body,
        grid=(grid,),
        in_specs=[
            yspec, yspec, yspec, yspec, cspec, cspec,
            pl.BlockSpec((4, DD, DD), lambda i: (0, 0, 0)),
            pl.BlockSpec((1, DD), lambda i: (0, 0)),
        ],
        out_specs=yspec,
        out_shape=jax.ShapeDtypeStruct((NN, DD), jnp.float32),
    )(y0, y1, y2, y3, disr, dis, w, b)


def _tail_body(y0_ref, y1_ref, y2_ref, y3_ref, disr_ref, w_ref, b_ref,
               oh_ref, w0_ref, b0_ref, w1_ref, b1_ref, out_ref,
               sum_s, mx_s, cnt_s):
    i = pl.program_id(0)

    @pl.when(i == 0)
    def _init():
        sum_s[...] = jnp.zeros((NG, DD), jnp.float32)
        mx_s[...] = jnp.full((NG, DD), -jnp.inf, jnp.float32)
        cnt_s[...] = jnp.zeros((NG, DD), jnp.float32)

    dr = disr_ref[...]
    w = w_ref[...]
    h = jnp.dot(y0_ref[...] * dr, w[0], preferred_element_type=jnp.float32)
    h += jnp.dot(y1_ref[...] * dr, w[1], preferred_element_type=jnp.float32)
    h += jnp.dot(y2_ref[...] * dr, w[2], preferred_element_type=jnp.float32)
    h += jnp.dot(y3_ref[...] * dr, w[3], preferred_element_type=jnp.float32)
    h += b_ref[...]
    oh = oh_ref[...]
    sum_s[...] += lax.dot_general(oh, h, (((0,), (0,)), ((), ())),
                                  preferred_element_type=jnp.float32)
    cnt_s[...] += jnp.sum(oh, axis=0)[:, None]
    mx = mx_s[...]
    mx_rows = []
    for g in range(NG):
        m = oh[:, g:g + 1] > 0.5
        mx_rows.append(jnp.max(jnp.where(m, h, -jnp.inf), axis=0)[None])
    mx_s[...] = jnp.maximum(mx, jnp.concatenate(mx_rows, axis=0))

    @pl.when(i == NN // BR - 1)
    def _finish():
        mean = sum_s[...] / jnp.maximum(cnt_s[...], 1.0)
        gcat = jnp.concatenate([mean, mx_s[...]], axis=1)
        gr = jnp.maximum(gcat, 0.0)
        a1 = jnp.maximum(
            jnp.dot(gr, w0_ref[...], preferred_element_type=jnp.float32)
            + b0_ref[...], 0.0)
        out_ref[...] = jnp.dot(
            a1, w1_ref[...], preferred_element_type=jnp.float32) + b1_ref[...]


def _tail_call(y0, y1, y2, y3, disr, w, b, onehot, w0, b0, w1, b1):
    grid = NN // BR
    yspec = pl.BlockSpec((BR, DD), lambda i: (i, 0))
    return pl.pallas_call(
        _tail_body,
        grid=(grid,),
        in_specs=[
            yspec, yspec, yspec, yspec,
            pl.BlockSpec((BR, 1), lambda i: (i, 0)),
            pl.BlockSpec((4, DD, DD), lambda i: (0, 0, 0)),
            pl.BlockSpec((1, DD), lambda i: (0, 0)),
            pl.BlockSpec((BR, NG), lambda i: (i, 0)),
            pl.BlockSpec((2 * DD, 2 * DD), lambda i: (0, 0)),
            pl.BlockSpec((1, 2 * DD), lambda i: (0, 0)),
            pl.BlockSpec((2 * DD, NCLS), lambda i: (0, 0)),
            pl.BlockSpec((1, NCLS), lambda i: (0, 0)),
        ],
        out_specs=pl.BlockSpec((NG, NCLS), lambda i: (0, 0)),
        out_shape=jax.ShapeDtypeStruct((NG, NCLS), jnp.float32),
        scratch_shapes=[
            pltpu.VMEM((NG, DD), jnp.float32),
            pltpu.VMEM((NG, DD), jnp.float32),
            pltpu.VMEM((NG, DD), jnp.float32),
        ],
    )(y0, y1, y2, y3, disr, w, b, onehot, w0, b0, w1, b1)


def kernel(x, edge_index, batch, conv0_w, conv0_b, conv1_w, conv1_b,
           mlp0_w, mlp0_b, pred_w, pred_b):
    row = edge_index[0]
    col = edge_index[1]

    ones_tab = jnp.ones((NN, DD), jnp.float32)
    rows_lin = jnp.tile(jnp.arange(EPW, dtype=jnp.int32), NWORK)
    degp = _sc_prop(ones_tab, rows_lin, col)

    dis, dis2, disr, y = _head_call(degp, x)

    onehot = (batch[:, None] == jnp.arange(NG, dtype=batch.dtype)
              ).astype(jnp.float32)

    ys = [y]
    for k in (1, 2, 3):
        pp = _sc_prop(ys[-1], row, col)
        ys.append(_tb_call(pp[0], pp[1], dis2))
    y = _mid_call(ys[0], ys[1], ys[2], ys[3], disr, dis, conv0_w,
                  conv0_b.reshape(1, DD))

    ys = [y]
    for k in (1, 2, 3):
        pp = _sc_prop(ys[-1], row, col)
        ys.append(_tb_call(pp[0], pp[1], dis2))
    return _tail_call(ys[0], ys[1], ys[2], ys[3], disr, conv1_w,
                      conv1_b.reshape(1, DD), onehot, mlp0_w,
                      mlp0_b.reshape(1, 2 * DD), pred_w,
                      pred_b.reshape(1, NCLS))
